# baseline (device time: 118361 ns/iter reference)
import jax
import jax.numpy as jnp
from jax import lax
from jax.experimental import pallas as pl
from jax.experimental.pallas import tpu as pltpu

N_Y = 4
B = 2
S_PER = 512
H = 8
D = 64
HD = H * D
NSUB = 2 * B
SCALE = D ** -0.5


def kernel(Q, K, V):
    def body(q_ref, k_ref, v_ref, out_ref,
             kvfull, send_r, recv_r, send_l, recv_l):
        my_x = lax.axis_index("x")
        my_y = lax.axis_index("y")
        my_z = lax.axis_index("z")
        has_r = my_y < N_Y - 1
        has_l = my_y > 0
        right = jnp.minimum(my_y + 1, N_Y - 1)
        left = jnp.maximum(my_y - 1, 0)

        barrier = pltpu.get_barrier_semaphore()

        @pl.when(has_l)
        def _():
            pl.semaphore_signal(
                barrier, inc=1,
                device_id=(my_x, left, my_z),
                device_id_type=pl.DeviceIdType.MESH,
            )

        @pl.when(has_r)
        def _():
            pl.semaphore_signal(
                barrier, inc=1,
                device_id=(my_x, right, my_z),
                device_id_type=pl.DeviceIdType.MESH,
            )

        pl.semaphore_wait(barrier, 1)

        @pl.when(has_l & has_r)
        def _():
            pl.semaphore_wait(barrier, 1)

        for p, (src, b) in enumerate([(k_ref, 0), (k_ref, 1),
                                      (v_ref, 0), (v_ref, 1)]):
            kvfull[pl.ds(my_y, 1), p] = (
                src[b].reshape(S_PER, HD).astype(jnp.bfloat16)[None]
            )

        def rdma(origin, p, dst_y, ssem, rsem):
            c = jnp.clip(origin, 0, N_Y - 1)
            return pltpu.make_async_remote_copy(
                src_ref=kvfull.at[c, p],
                dst_ref=kvfull.at[c, p],
                send_sem=ssem.at[c, p],
                recv_sem=rsem.at[c, p],
                device_id=(my_x, dst_y, my_z),
                device_id_type=pl.DeviceIdType.MESH,
            )

        @pl.when(has_r)
        def _():
            for p in range(NSUB):
                rdma(my_y, p, right, send_r, recv_r).start()

        @pl.when(has_l)
        def _():
            for p in range(NSUB):
                rdma(my_y, p, left, send_l, recv_l).start()

        for i in range(NSUB * (N_Y - 1)):
            j, p = divmod(i, NSUB)
            c_r = my_y - 1 - j
            c_l = my_y + 1 + j

            @pl.when(c_r >= 0)
            def _(c_r=c_r, p=p):
                rdma(c_r, p, left, send_r, recv_r).wait_recv()

            @pl.when((c_r >= 0) & has_r)
            def _(c_r=c_r, p=p):
                rdma(c_r, p, right, send_r, recv_r).start()

            @pl.when(c_l <= N_Y - 1)
            def _(c_l=c_l, p=p):
                rdma(c_l, p, right, send_l, recv_l).wait_recv()

            @pl.when((c_l <= N_Y - 1) & has_l)
            def _(c_l=c_l, p=p):
                rdma(c_l, p, left, send_l, recv_l).start()

        for j in range(N_Y - 1):
            @pl.when(has_r & (my_y - j >= 0))
            def _(j=j):
                for p in range(NSUB):
                    rdma(my_y - j, p, right, send_r, recv_r).wait_send()

            @pl.when(has_l & (my_y + j <= N_Y - 1))
            def _(j=j):
                for p in range(NSUB):
                    rdma(my_y + j, p, left, send_l, recv_l).wait_send()

        for b in range(B):
            for hh in range(H):
                q = q_ref[b, :, hh, :].astype(jnp.bfloat16)
                o = None
                l = None
                for c in range(N_Y):
                    kc = kvfull[c, b, :, hh * D:(hh + 1) * D]
                    s = lax.dot_general(
                        q, kc, (((1,), (1,)), ((), ())),
                        preferred_element_type=jnp.float32,
                    ) * SCALE
                    pexp = jnp.exp(s)
                    lc = jnp.sum(pexp, axis=1, keepdims=True)
                    vc = kvfull[c, B + b, :, hh * D:(hh + 1) * D]
                    oc = lax.dot_general(
                        pexp.astype(jnp.bfloat16), vc,
                        (((1,), (0,)), ((), ())),
                        preferred_element_type=jnp.float32,
                    )
                    o = oc if o is None else o + oc
                    l = lc if l is None else l + lc
                out_ref[b, :, hh, :] = o / l

    return pl.pallas_call(
        body,
        out_shape=jax.ShapeDtypeStruct((B, S_PER, H, D), jnp.float32),
        in_specs=[
            pl.BlockSpec(memory_space=pltpu.VMEM),
            pl.BlockSpec(memory_space=pltpu.VMEM),
            pl.BlockSpec(memory_space=pltpu.VMEM),
        ],
        out_specs=pl.BlockSpec(memory_space=pltpu.VMEM),
        scratch_shapes=[
            pltpu.VMEM((N_Y, NSUB, S_PER, HD), jnp.bfloat16),
            pltpu.SemaphoreType.DMA((N_Y, NSUB)),
            pltpu.SemaphoreType.DMA((N_Y, NSUB)),
            pltpu.SemaphoreType.DMA((N_Y, NSUB)),
            pltpu.SemaphoreType.DMA((N_Y, NSUB)),
        ],
        compiler_params=pltpu.CompilerParams(
            collective_id=0,
            vmem_limit_bytes=60 * 1024 * 1024,
        ),
    )(Q, K, V)
